# baseline (speedup 1.0000x reference)
import functools

import jax
import jax.numpy as jnp
from jax import lax
from jax.experimental import pallas as pl
from jax.experimental.pallas import tpu as pltpu
from jax.experimental.pallas import tpu_sc as plsc

N = 10000
E = 320000
D = 128

NC = 2
NS = 16
NW = NC * NS

CHUNK = 125
NITER = 80
NHALF = NITER // 2
E_PER_W = CHUNK * NITER
E_PAD = E_PER_W * NW
N_PAD = 10112
STRIPE = N_PAD // NS
DST_PAD = N


_sc_mesh = plsc.VectorSubcoreMesh(core_axis_name="c", subcore_axis_name="s")


@functools.partial(
    pl.kernel,
    out_type=jax.ShapeDtypeStruct((NC, N_PAD, D), jnp.float32),
    mesh=_sc_mesh,
    scratch_types=[
        pltpu.VMEM((NHALF, CHUNK), jnp.int32),
        pltpu.VMEM((NHALF, CHUNK), jnp.int32),
        pltpu.VMEM((CHUNK, D), jnp.float32),
        pltpu.VMEM((CHUNK, D), jnp.float32),
        pltpu.VMEM_SHARED((N_PAD, D), jnp.float32),
        pltpu.SemaphoreType.DMA,
        pltpu.SemaphoreType.DMA,
        pltpu.SemaphoreType.DMA,
        pltpu.SemaphoreType.DMA,
    ],
)
def _sc_aggregate(x_hbm, src_hbm, dst_hbm, zeros_hbm, out_hbm,
                  src_v, dst_v, rows0, rows1, agg_sh, gsem0, gsem1,
                  ssem0, ssem1):
    cid = lax.axis_index("c")
    sid = lax.axis_index("s")
    wid = sid * NC + cid

    pltpu.sync_copy(zeros_hbm, agg_sh.at[pl.ds(sid * STRIPE, STRIPE)])
    plsc.subcore_barrier()

    def wait_scatter(buf, sem):
        pltpu.make_async_copy(buf, agg_sh.at[dst_v.at[0]], sem).wait()

    for h in range(2):
        pltpu.sync_copy(src_hbm.at[wid, pl.ds(h * NHALF, NHALF)], src_v)
        pltpu.sync_copy(dst_hbm.at[wid, pl.ds(h * NHALF, NHALF)], dst_v)
        pltpu.async_copy(x_hbm.at[src_v.at[0]], rows0, gsem0)

        def body(k, _):
            i0 = 2 * k
            i1 = 2 * k + 1
            i2 = 2 * k + 2
            pltpu.make_async_copy(x_hbm.at[src_v.at[i0]], rows0, gsem0).wait()
            pltpu.async_copy(rows0, agg_sh.at[dst_v.at[i0]], ssem0, add=True)

            @pl.when(k > 0)
            def _():
                wait_scatter(rows1, ssem1)

            pltpu.async_copy(x_hbm.at[src_v.at[i1]], rows1, gsem1)
            pltpu.make_async_copy(x_hbm.at[src_v.at[i1]], rows1, gsem1).wait()
            pltpu.async_copy(rows1, agg_sh.at[dst_v.at[i1]], ssem1, add=True)

            @pl.when(i2 < NHALF)
            def _():
                wait_scatter(rows0, ssem0)
                pltpu.async_copy(x_hbm.at[src_v.at[i2]], rows0, gsem0)

            return ()

        lax.fori_loop(0, NHALF // 2, body, (), unroll=False)
        wait_scatter(rows0, ssem0)
        wait_scatter(rows1, ssem1)

    plsc.subcore_barrier()
    pltpu.sync_copy(
        agg_sh.at[pl.ds(sid * STRIPE, STRIPE)],
        out_hbm.at[cid, pl.ds(sid * STRIPE, STRIPE)],
    )


_BLK = 632


def _tc_linear_body(agg_ref, w_ref, b_ref, o_ref):
    a = agg_ref[0] + agg_ref[1]
    h = lax.dot_general(a, w_ref[...], (((1,), (1,)), ((), ())),
                        preferred_element_type=jnp.float32)
    o_ref[...] = jnp.maximum(h + b_ref[...], 0.0)


def _tc_linear(agg2, W, b):
    return pl.pallas_call(
        _tc_linear_body,
        grid=(N_PAD // _BLK,),
        in_specs=[
            pl.BlockSpec((NC, _BLK, D), lambda i: (0, i, 0)),
            pl.BlockSpec((D, D), lambda i: (0, 0)),
            pl.BlockSpec((1, D), lambda i: (0, 0)),
        ],
        out_specs=pl.BlockSpec((_BLK, D), lambda i: (i, 0)),
        out_shape=jax.ShapeDtypeStruct((N_PAD, D), jnp.float32),
    )(agg2, W, b.reshape(1, D))


def kernel(x, edge_index, W, b):
    ei = edge_index.astype(jnp.int32)
    pad = E_PAD - E
    pad_dst = DST_PAD + jnp.arange(pad, dtype=jnp.int32) % (N_PAD - N)
    src = jnp.concatenate([ei[0], jnp.zeros((pad,), jnp.int32)])
    dst = jnp.concatenate([ei[1], pad_dst])
    src = src.reshape(NW, NITER, CHUNK)
    dst = dst.reshape(NW, NITER, CHUNK)
    zeros = jnp.zeros((STRIPE, D), jnp.float32)
    agg2 = _sc_aggregate(x, src, dst, zeros)
    return _tc_linear(agg2, W, b)[:N]


# speedup vs baseline: 1.0016x; 1.0016x over previous
import functools

import jax
import jax.numpy as jnp
from jax import lax
from jax.experimental import pallas as pl
from jax.experimental.pallas import tpu as pltpu
from jax.experimental.pallas import tpu_sc as plsc

N = 10000
E = 320000
D = 128

NC = 2
NS = 16
NW = NC * NS

CHUNK = 125
NITER = 80
NHALF = NITER // 2
E_PER_W = CHUNK * NITER
E_PAD = E_PER_W * NW
N_PAD = 10112
STRIPE = N_PAD // NS
DST_PAD = N


_sc_mesh = plsc.VectorSubcoreMesh(core_axis_name="c", subcore_axis_name="s")


@functools.partial(
    pl.kernel,
    out_type=jax.ShapeDtypeStruct((NC, N_PAD, D), jnp.float32),
    mesh=_sc_mesh,
    scratch_types=[
        pltpu.VMEM((NHALF, CHUNK), jnp.int32),
        pltpu.VMEM((NHALF, CHUNK), jnp.int32),
        pltpu.VMEM((CHUNK, D), jnp.float32),
        pltpu.VMEM((CHUNK, D), jnp.float32),
        pltpu.VMEM_SHARED((N_PAD, D), jnp.float32),
        pltpu.SemaphoreType.DMA,
        pltpu.SemaphoreType.DMA,
        pltpu.SemaphoreType.DMA,
        pltpu.SemaphoreType.DMA,
    ],
)
def _sc_aggregate(x_hbm, src_hbm, dst_hbm, zeros_hbm, out_hbm,
                  src_v, dst_v, rows0, rows1, agg_sh, gsem0, gsem1,
                  ssem0, ssem1):
    cid = lax.axis_index("c")
    sid = lax.axis_index("s")
    wid = sid * NC + cid

    pltpu.sync_copy(zeros_hbm, agg_sh.at[pl.ds(sid * STRIPE, STRIPE)])
    plsc.subcore_barrier()

    def wait_scatter(buf, sem):
        pltpu.make_async_copy(buf, agg_sh.at[dst_v.at[0]], sem).wait()

    for h in range(2):
        pltpu.sync_copy(src_hbm.at[wid, pl.ds(h * NHALF, NHALF)], src_v)
        pltpu.sync_copy(dst_hbm.at[wid, pl.ds(h * NHALF, NHALF)], dst_v)
        pltpu.async_copy(x_hbm.at[src_v.at[0]], rows0, gsem0)

        def body(k, _):
            i0 = 2 * k
            i1 = 2 * k + 1
            i2 = 2 * k + 2
            pltpu.make_async_copy(x_hbm.at[src_v.at[i0]], rows0, gsem0).wait()
            pltpu.async_copy(rows0, agg_sh.at[dst_v.at[i0]], ssem0, add=True)

            @pl.when(k > 0)
            def _():
                wait_scatter(rows1, ssem1)

            pltpu.async_copy(x_hbm.at[src_v.at[i1]], rows1, gsem1)
            pltpu.make_async_copy(x_hbm.at[src_v.at[i1]], rows1, gsem1).wait()
            pltpu.async_copy(rows1, agg_sh.at[dst_v.at[i1]], ssem1, add=True)

            @pl.when(i2 < NHALF)
            def _():
                wait_scatter(rows0, ssem0)
                pltpu.async_copy(x_hbm.at[src_v.at[i2]], rows0, gsem0)

            return ()

        lax.fori_loop(0, NHALF // 2, body, (), unroll=False)
        wait_scatter(rows0, ssem0)
        wait_scatter(rows1, ssem1)

    plsc.subcore_barrier()
    pltpu.sync_copy(
        agg_sh.at[pl.ds(sid * STRIPE, STRIPE)],
        out_hbm.at[cid, pl.ds(sid * STRIPE, STRIPE)],
    )


_BLK = 632


def _tc_linear_body(agg_ref, w_ref, b_ref, o_ref):
    a = agg_ref[0] + agg_ref[1]
    h = lax.dot_general(a, w_ref[...], (((1,), (1,)), ((), ())),
                        preferred_element_type=jnp.float32)
    o_ref[...] = jnp.maximum(h + b_ref[...], 0.0)


def _tc_linear(agg2, W, b):
    return pl.pallas_call(
        _tc_linear_body,
        grid=(N_PAD // _BLK,),
        in_specs=[
            pl.BlockSpec((NC, _BLK, D), lambda i: (0, i, 0)),
            pl.BlockSpec((D, D), lambda i: (0, 0)),
            pl.BlockSpec((1, D), lambda i: (0, 0)),
        ],
        out_specs=pl.BlockSpec((_BLK, D), lambda i: (i, 0)),
        out_shape=jax.ShapeDtypeStruct((N_PAD, D), jnp.float32),
    )(agg2, W, b.reshape(1, D))


def kernel(x, edge_index, W, b):
    ei = edge_index.astype(jnp.int32)
    src = ei[0].reshape(NW, NITER, CHUNK)
    dst = ei[1].reshape(NW, NITER, CHUNK)
    zeros = jnp.zeros((STRIPE, D), jnp.float32)
    agg2 = _sc_aggregate(x, src, dst, zeros)
    return _tc_linear(agg2, W, b)[:N]


# speedup vs baseline: 1.0429x; 1.0412x over previous
import functools

import jax
import jax.numpy as jnp
from jax import lax
from jax.experimental import pallas as pl
from jax.experimental.pallas import tpu as pltpu
from jax.experimental.pallas import tpu_sc as plsc

N = 10000
E = 320000
D = 128

NC = 2
NS = 16
NW = NC * NS

CHUNK = 125
NITER = 80
NHALF = NITER // 2
E_PER_W = CHUNK * NITER
E_PAD = E_PER_W * NW
N_PAD = 10112
STRIPE = N_PAD // NS
DST_PAD = N


_sc_mesh = plsc.VectorSubcoreMesh(core_axis_name="c", subcore_axis_name="s")


@functools.partial(
    pl.kernel,
    out_type=jax.ShapeDtypeStruct((NC, N_PAD, D), jnp.float32),
    mesh=_sc_mesh,
    scratch_types=[
        pltpu.VMEM((NHALF, CHUNK), jnp.int32),
        pltpu.VMEM((NHALF, CHUNK), jnp.int32),
        pltpu.VMEM((CHUNK, D), jnp.float32),
        pltpu.VMEM((CHUNK, D), jnp.float32),
        pltpu.VMEM_SHARED((N_PAD, D), jnp.float32),
        pltpu.SemaphoreType.DMA,
        pltpu.SemaphoreType.DMA,
        pltpu.SemaphoreType.DMA,
        pltpu.SemaphoreType.DMA,
    ],
)
def _sc_aggregate(x_hbm, src_hbm, dst_hbm, zeros_hbm, out_hbm,
                  src_v, dst_v, rows0, rows1, agg_sh, gsem0, gsem1,
                  ssem0, ssem1):
    cid = lax.axis_index("c")
    sid = lax.axis_index("s")
    wid = sid * NC + cid

    pltpu.sync_copy(zeros_hbm, agg_sh.at[pl.ds(sid * STRIPE, STRIPE)])
    plsc.subcore_barrier()

    def wait_scatter(buf, sem):
        pltpu.make_async_copy(buf, agg_sh.at[dst_v.at[0]], sem).wait()

    for h in range(2):
        pltpu.sync_copy(src_hbm.at[wid, pl.ds(h * NHALF, NHALF)], src_v)
        pltpu.sync_copy(dst_hbm.at[wid, pl.ds(h * NHALF, NHALF)], dst_v)
        pltpu.async_copy(x_hbm.at[src_v.at[0]], rows0, gsem0)

        def body(k, _):
            i0 = 2 * k
            i1 = 2 * k + 1
            i2 = 2 * k + 2
            pltpu.make_async_copy(x_hbm.at[src_v.at[i0]], rows0, gsem0).wait()
            pltpu.async_copy(rows0, agg_sh.at[dst_v.at[i0]], ssem0, add=True)

            @pl.when(k > 0)
            def _():
                wait_scatter(rows1, ssem1)

            pltpu.async_copy(x_hbm.at[src_v.at[i1]], rows1, gsem1)
            pltpu.make_async_copy(x_hbm.at[src_v.at[i1]], rows1, gsem1).wait()
            pltpu.async_copy(rows1, agg_sh.at[dst_v.at[i1]], ssem1, add=True)

            @pl.when(i2 < NHALF)
            def _():
                wait_scatter(rows0, ssem0)
                pltpu.async_copy(x_hbm.at[src_v.at[i2]], rows0, gsem0)

            return ()

        lax.fori_loop(0, NHALF // 2, body, (), unroll=False)
        wait_scatter(rows0, ssem0)
        wait_scatter(rows1, ssem1)

    plsc.subcore_barrier()
    pltpu.sync_copy(
        agg_sh.at[pl.ds(sid * STRIPE, STRIPE)],
        out_hbm.at[cid, pl.ds(sid * STRIPE, STRIPE)],
    )


_BLK = 10112


def _tc_linear_body(agg_ref, w_ref, b_ref, o_ref):
    a = agg_ref[0] + agg_ref[1]
    h = lax.dot_general(a, w_ref[...], (((1,), (1,)), ((), ())),
                        preferred_element_type=jnp.float32)
    o_ref[...] = jnp.maximum(h + b_ref[...], 0.0)


def _tc_linear(agg2, W, b):
    return pl.pallas_call(
        _tc_linear_body,
        grid=(N_PAD // _BLK,),
        in_specs=[
            pl.BlockSpec((NC, _BLK, D), lambda i: (0, i, 0)),
            pl.BlockSpec((D, D), lambda i: (0, 0)),
            pl.BlockSpec((1, D), lambda i: (0, 0)),
        ],
        out_specs=pl.BlockSpec((_BLK, D), lambda i: (i, 0)),
        out_shape=jax.ShapeDtypeStruct((N_PAD, D), jnp.float32),
    )(agg2, W, b.reshape(1, D))


def kernel(x, edge_index, W, b):
    ei = edge_index.astype(jnp.int32)
    src = ei[0].reshape(NW, NITER, CHUNK)
    dst = ei[1].reshape(NW, NITER, CHUNK)
    zeros = jnp.zeros((STRIPE, D), jnp.float32)
    agg2 = _sc_aggregate(x, src, dst, zeros)
    return _tc_linear(agg2, W, b)[:N]


# speedup vs baseline: 1.0652x; 1.0214x over previous
import functools

import jax
import jax.numpy as jnp
from jax import lax
from jax.experimental import pallas as pl
from jax.experimental.pallas import tpu as pltpu
from jax.experimental.pallas import tpu_sc as plsc

N = 10000
E = 320000
D = 128

NC = 2
NS = 16
NW = NC * NS

CHUNK = 125
NITER = 80
NHALF = NITER // 2
E_PER_W = CHUNK * NITER
E_PAD = E_PER_W * NW
N_PAD = 10112
STRIPE = N_PAD // NS
DST_PAD = N


_sc_mesh = plsc.VectorSubcoreMesh(core_axis_name="c", subcore_axis_name="s")


@functools.partial(
    pl.kernel,
    out_type=jax.ShapeDtypeStruct((NC, N_PAD, D), jnp.float32),
    mesh=_sc_mesh,
    scratch_types=[
        pltpu.VMEM((NHALF, CHUNK), jnp.int32),
        pltpu.VMEM((NHALF, CHUNK), jnp.int32),
        pltpu.VMEM((CHUNK, D), jnp.float32),
        pltpu.VMEM((CHUNK, D), jnp.float32),
        pltpu.VMEM_SHARED((N_PAD, D), jnp.float32),
        pltpu.SemaphoreType.DMA,
        pltpu.SemaphoreType.DMA,
        pltpu.SemaphoreType.DMA,
        pltpu.SemaphoreType.DMA,
    ],
)
def _sc_aggregate(x_hbm, src_hbm, dst_hbm, zeros_hbm, out_hbm,
                  src_v, dst_v, rows0, rows1, agg_sh, gsem0, gsem1,
                  ssem0, ssem1):
    cid = lax.axis_index("c")
    sid = lax.axis_index("s")
    wid = sid * NC + cid

    pltpu.sync_copy(zeros_hbm, agg_sh.at[pl.ds(sid * STRIPE, STRIPE)])
    plsc.subcore_barrier()

    def wait_scatter(buf, sem):
        pltpu.make_async_copy(buf, agg_sh.at[dst_v.at[0]], sem).wait()

    for h in range(2):
        pltpu.sync_copy(src_hbm.at[wid, pl.ds(h * NHALF, NHALF)], src_v)
        pltpu.sync_copy(dst_hbm.at[wid, pl.ds(h * NHALF, NHALF)], dst_v)
        pltpu.async_copy(x_hbm.at[src_v.at[0]], rows0, gsem0)

        def body(k, _):
            i0 = 2 * k
            i1 = 2 * k + 1
            i2 = 2 * k + 2
            pltpu.make_async_copy(x_hbm.at[src_v.at[i0]], rows0, gsem0).wait()
            pltpu.async_copy(rows0, agg_sh.at[dst_v.at[i0]], ssem0, add=True)

            @pl.when(k > 0)
            def _():
                wait_scatter(rows1, ssem1)

            pltpu.async_copy(x_hbm.at[src_v.at[i1]], rows1, gsem1)
            pltpu.make_async_copy(x_hbm.at[src_v.at[i1]], rows1, gsem1).wait()
            pltpu.async_copy(rows1, agg_sh.at[dst_v.at[i1]], ssem1, add=True)

            @pl.when(i2 < NHALF)
            def _():
                wait_scatter(rows0, ssem0)
                pltpu.async_copy(x_hbm.at[src_v.at[i2]], rows0, gsem0)

            return ()

        lax.fori_loop(0, NHALF // 2, body, (), unroll=False)
        wait_scatter(rows0, ssem0)
        wait_scatter(rows1, ssem1)

    plsc.subcore_barrier()
    pltpu.sync_copy(
        agg_sh.at[pl.ds(sid * STRIPE, STRIPE)],
        out_hbm.at[cid, pl.ds(sid * STRIPE, STRIPE)],
    )


def _tc_linear_body(agg_ref, w_ref, b_ref, o_ref):
    a = agg_ref[0, :N] + agg_ref[1, :N]
    h = lax.dot_general(a, w_ref[...], (((1,), (1,)), ((), ())),
                        preferred_element_type=jnp.float32)
    o_ref[...] = jnp.maximum(h + b_ref[...], 0.0)


def _tc_linear(agg2, W, b):
    return pl.pallas_call(
        _tc_linear_body,
        grid=(1,),
        in_specs=[
            pl.BlockSpec((NC, N_PAD, D), lambda i: (0, 0, 0)),
            pl.BlockSpec((D, D), lambda i: (0, 0)),
            pl.BlockSpec((1, D), lambda i: (0, 0)),
        ],
        out_specs=pl.BlockSpec((N, D), lambda i: (0, 0)),
        out_shape=jax.ShapeDtypeStruct((N, D), jnp.float32),
    )(agg2, W, b.reshape(1, D))


def kernel(x, edge_index, W, b):
    ei = edge_index.astype(jnp.int32)
    src = ei[0].reshape(NW, NITER, CHUNK)
    dst = ei[1].reshape(NW, NITER, CHUNK)
    zeros = jnp.zeros((STRIPE, D), jnp.float32)
    agg2 = _sc_aggregate(x, src, dst, zeros)
    return _tc_linear(agg2, W, b)


# speedup vs baseline: 1.0766x; 1.0107x over previous
import functools

import jax
import jax.numpy as jnp
from jax import lax
from jax.experimental import pallas as pl
from jax.experimental.pallas import tpu as pltpu
from jax.experimental.pallas import tpu_sc as plsc

N = 10000
E = 320000
D = 128

NC = 2
NS = 16
NW = NC * NS

CHUNK = 125
NITER = 80
NHALF = NITER // 2
E_PER_W = CHUNK * NITER
E_PAD = E_PER_W * NW
N_PAD = 10112
STRIPE = N_PAD // NS
DST_PAD = N


_sc_mesh = plsc.VectorSubcoreMesh(core_axis_name="c", subcore_axis_name="s")


@functools.partial(
    pl.kernel,
    out_type=jax.ShapeDtypeStruct((NC, N_PAD, D), jnp.float32),
    mesh=_sc_mesh,
    scratch_types=[
        pltpu.VMEM((NHALF, CHUNK), jnp.int32),
        pltpu.VMEM((NHALF, CHUNK), jnp.int32),
        pltpu.VMEM((CHUNK, D), jnp.float32),
        pltpu.VMEM((CHUNK, D), jnp.float32),
        pltpu.VMEM_SHARED((N_PAD, D), jnp.float32),
        pltpu.SemaphoreType.DMA,
        pltpu.SemaphoreType.DMA,
        pltpu.SemaphoreType.DMA,
        pltpu.SemaphoreType.DMA,
    ],
)
def _sc_aggregate(x_hbm, src_hbm, dst_hbm, zeros_hbm, out_hbm,
                  src_v, dst_v, rows0, rows1, agg_sh, gsem0, gsem1,
                  ssem0, ssem1):
    cid = lax.axis_index("c")
    sid = lax.axis_index("s")
    wid = sid * NC + cid

    pltpu.async_copy(zeros_hbm, agg_sh.at[pl.ds(sid * STRIPE, STRIPE)], ssem0)

    def wait_scatter(buf, sem):
        pltpu.make_async_copy(buf, agg_sh.at[dst_v.at[0]], sem).wait()

    for h in range(2):
        pltpu.sync_copy(src_hbm.at[wid, pl.ds(h * NHALF, NHALF)], src_v)
        pltpu.sync_copy(dst_hbm.at[wid, pl.ds(h * NHALF, NHALF)], dst_v)
        pltpu.async_copy(x_hbm.at[src_v.at[0]], rows0, gsem0)
        if h == 0:
            pltpu.make_async_copy(
                zeros_hbm, agg_sh.at[pl.ds(sid * STRIPE, STRIPE)], ssem0
            ).wait()
            plsc.subcore_barrier()

        def body(k, _):
            i0 = 2 * k
            i1 = 2 * k + 1
            i2 = 2 * k + 2
            pltpu.make_async_copy(x_hbm.at[src_v.at[i0]], rows0, gsem0).wait()
            pltpu.async_copy(rows0, agg_sh.at[dst_v.at[i0]], ssem0, add=True)

            @pl.when(k > 0)
            def _():
                wait_scatter(rows1, ssem1)

            pltpu.async_copy(x_hbm.at[src_v.at[i1]], rows1, gsem1)
            pltpu.make_async_copy(x_hbm.at[src_v.at[i1]], rows1, gsem1).wait()
            pltpu.async_copy(rows1, agg_sh.at[dst_v.at[i1]], ssem1, add=True)

            @pl.when(i2 < NHALF)
            def _():
                wait_scatter(rows0, ssem0)
                pltpu.async_copy(x_hbm.at[src_v.at[i2]], rows0, gsem0)

            return ()

        lax.fori_loop(0, NHALF // 2, body, (), unroll=False)
        wait_scatter(rows0, ssem0)
        wait_scatter(rows1, ssem1)

    plsc.subcore_barrier()
    pltpu.sync_copy(
        agg_sh.at[pl.ds(sid * STRIPE, STRIPE)],
        out_hbm.at[cid, pl.ds(sid * STRIPE, STRIPE)],
    )


def _tc_linear_body(agg_ref, w_ref, b_ref, o_ref):
    a = agg_ref[0, :N] + agg_ref[1, :N]
    h = lax.dot_general(a, w_ref[...], (((1,), (1,)), ((), ())),
                        preferred_element_type=jnp.float32)
    o_ref[...] = jnp.maximum(h + b_ref[...], 0.0)


def _tc_linear(agg2, W, b):
    return pl.pallas_call(
        _tc_linear_body,
        grid=(1,),
        in_specs=[
            pl.BlockSpec((NC, N_PAD, D), lambda i: (0, 0, 0)),
            pl.BlockSpec((D, D), lambda i: (0, 0)),
            pl.BlockSpec((1, D), lambda i: (0, 0)),
        ],
        out_specs=pl.BlockSpec((N, D), lambda i: (0, 0)),
        out_shape=jax.ShapeDtypeStruct((N, D), jnp.float32),
    )(agg2, W, b.reshape(1, D))


def kernel(x, edge_index, W, b):
    ei = edge_index.astype(jnp.int32)
    src = ei[0].reshape(NW, NITER, CHUNK)
    dst = ei[1].reshape(NW, NITER, CHUNK)
    zeros = jnp.zeros((STRIPE, D), jnp.float32)
    agg2 = _sc_aggregate(x, src, dst, zeros)
    return _tc_linear(agg2, W, b)
